# Initial kernel scaffold; baseline (speedup 1.0000x reference)
import jax
import jax.numpy as jnp
from jax.experimental import pallas as pl


def kernel(logits, labels):
    raise NotImplementedError("write your pallas kernel here")


# speedup vs baseline: 4.7704x; 4.7704x over previous
import functools

import jax
import jax.numpy as jnp
from jax import lax
from jax.experimental import pallas as pl
from jax.experimental.pallas import tpu as pltpu
from jax.experimental.pallas import tpu_sc as plsc

IGNORE = 255
KEEP_RATIO = 0.3
MIN_KEPT = 100000


_RB = 256


def _ce_body(lg_ref, lb_ref, out_ref):
    x = lg_ref[0]
    m = jnp.max(x, axis=0)
    e = jnp.exp(x - m[None])
    s = jnp.sum(e, axis=0)
    lse = jnp.log(s) + m
    lbl = lb_ref[0]
    ids = lax.broadcasted_iota(jnp.int32, x.shape, 0)
    xl = jnp.sum(jnp.where(ids == lbl[None], x, 0.0), axis=0)
    out_ref[0] = lse - xl


def _ce_loss(logits, labels):
    B, C, H, W = logits.shape
    rows = H * W // 128
    lg = logits.reshape(B, C, rows, 128)
    lb = labels.reshape(B, rows, 128)
    grid = (B, rows // _RB)
    out = pl.pallas_call(
        _ce_body,
        grid=grid,
        in_specs=[
            pl.BlockSpec((1, C, _RB, 128), lambda b, r: (b, 0, r, 0)),
            pl.BlockSpec((1, _RB, 128), lambda b, r: (b, r, 0)),
        ],
        out_specs=pl.BlockSpec((1, _RB, 128), lambda b, r: (b, r, 0)),
        out_shape=jax.ShapeDtypeStruct((B, rows, 128), jnp.float32),
        compiler_params=pltpu.CompilerParams(
            dimension_semantics=("parallel", "parallel")),
    )(lg, lb)
    return out.reshape(B * H * W)


_T = 16
_CH = 8192


def _sc_select(loss, n, k):
    per_tile = n // _T
    nch = per_tile // _CH
    nv = _CH // 16
    kf = float(k)

    mesh = plsc.VectorSubcoreMesh(
        core_axis_name="c", subcore_axis_name="s", num_cores=1)

    @functools.partial(
        pl.kernel,
        out_type=jax.ShapeDtypeStruct((16,), jnp.float32),
        mesh=mesh,
        compiler_params=pltpu.CompilerParams(needs_layout_passes=False),
        scratch_types=[
            pltpu.VMEM((_CH,), jnp.float32),
            pltpu.VMEM((4096,), jnp.float32),
            pltpu.VMEM((16, 256), jnp.float32),
            pltpu.VMEM((16,), jnp.float32),
            pltpu.VMEM((256,), jnp.float32),
            pltpu.VMEM((16,), jnp.float32),
            pltpu.VMEM((16,), jnp.float32),
            pltpu.VMEM_SHARED((16, 4096), jnp.float32),
            pltpu.VMEM_SHARED((256,), jnp.float32),
        ],
    )
    def sel(loss_hbm, out_hbm, buf, hist, mbuf, stage, totb, asum, acnt,
            sh_hist, sh_tot):
        sid = lax.axis_index("s")
        lane = lax.broadcasted_iota(jnp.int32, (16,), 0)
        ones = jnp.ones((16,), jnp.float32)
        zeros = jnp.zeros((16,), jnp.float32)

        prefix = jnp.uint32(0)
        k_rem = jnp.float32(kf)

        for lvl in range(4):
            sh = 24 - 8 * lvl

            def zb(i, _):
                hist[pl.ds(i * 16, 16)] = zeros
                return 0
            lax.fori_loop(0, 256, zb, 0)

            pfx_hi = prefix >> (sh + 8) if lvl > 0 else None

            def chunk(c, _):
                base = (sid * per_tile + c * _CH).astype(jnp.int32)
                pltpu.sync_copy(loss_hbm.at[pl.ds(base, _CH)], buf)

                def vec(j, _):
                    v = buf[pl.ds(j * 16, 16)]
                    u = lax.bitcast_convert_type(v, jnp.uint32)
                    b = ((u >> sh) & 0xFF).astype(jnp.int32)
                    idx = b * 16 + lane
                    if lvl == 0:
                        plsc.addupdate_scatter(hist, [idx], ones)
                    else:
                        pm = (u >> (sh + 8)) == pfx_hi
                        plsc.addupdate_scatter(hist, [idx], ones, mask=pm)
                    return 0
                lax.fori_loop(0, nv, vec, 0)
                return 0
            lax.fori_loop(0, nch, chunk, 0)

            pltpu.sync_copy(hist, sh_hist.at[sid])
            plsc.subcore_barrier()
            for src in range(16):
                pltpu.sync_copy(sh_hist.at[src, pl.ds(sid * 256, 256)],
                                mbuf.at[src])
            tvec = zeros
            for bb in range(16):
                acc = zeros
                for src in range(16):
                    acc = acc + mbuf[src, pl.ds(bb * 16, 16)]
                tvec = jnp.where(lane == bb, jnp.sum(acc), tvec)
            stage[...] = tvec
            pltpu.sync_copy(stage, sh_tot.at[pl.ds(sid * 16, 16)])
            plsc.subcore_barrier()
            pltpu.sync_copy(sh_tot, totb)

            cum = jnp.float32(0.0)
            bstar = jnp.float32(-1.0)
            above = jnp.float32(0.0)
            for i in range(16):
                g = 15 - i
                c = totb[pl.ds(g * 16, 16)]
                r = lax.rev(c, (0,))
                cr = jnp.cumsum(r)
                mrk = (cum + cr) >= k_rem
                pc = jnp.max(plsc.all_reduce_population_count(mrk))
                jstar = 16 - pc
                sel_v = jnp.where(lane == jstar, 1.0, 0.0)
                crj = jnp.sum(cr * sel_v)
                rj = jnp.sum(r * sel_v)
                cum_new = cum + jnp.sum(c)
                cond = (bstar < 0.0) & (cum_new >= k_rem)
                bval = (g * 16 + 15 - jstar).astype(jnp.float32)
                bstar = jnp.where(cond, bval, bstar)
                above = jnp.where(cond, cum + crj - rj, above)
                cum = cum_new
            prefix = prefix | (bstar.astype(jnp.uint32) << sh)
            k_rem = k_rem - above
            plsc.subcore_barrier()

        asum[...] = zeros
        acnt[...] = zeros

        def chunk2(c, _):
            base = (sid * per_tile + c * _CH).astype(jnp.int32)
            pltpu.sync_copy(loss_hbm.at[pl.ds(base, _CH)], buf)

            def vec(j, _):
                v = buf[pl.ds(j * 16, 16)]
                u = lax.bitcast_convert_type(v, jnp.uint32)
                mk = u >= prefix
                asum[...] = asum[...] + jnp.where(mk, v, 0.0)
                acnt[...] = acnt[...] + jnp.where(mk, ones, zeros)
                return 0
            lax.fori_loop(0, nv, vec, 0)
            return 0
        lax.fori_loop(0, nch, chunk2, 0)

        ts = jnp.sum(asum[...])
        tc = jnp.sum(acnt[...])
        stage[...] = jnp.where(lane == 0, ts, jnp.where(lane == 1, tc, 0.0))
        pltpu.sync_copy(stage, sh_tot.at[pl.ds(sid * 16, 16)])
        plsc.subcore_barrier()

        @pl.when(sid == 0)
        def _():
            pltpu.sync_copy(sh_tot, totb)
            acc = zeros
            for t in range(16):
                acc = acc + totb[pl.ds(t * 16, 16)]
            s = jnp.sum(jnp.where(lane == 0, acc, 0.0))
            cnt = jnp.sum(jnp.where(lane == 1, acc, 0.0))
            stage[...] = (ones * s) / (ones * cnt)
            pltpu.sync_copy(stage, out_hbm)

    return sel(loss)


def kernel(logits, labels):
    B, C, H, W = logits.shape
    n = B * H * W
    k = min(max(int(KEEP_RATIO * n), min(MIN_KEPT, n)), n)
    loss = _ce_loss(logits, labels)
    out16 = _sc_select(loss, n, k)
    return out16[0]


# speedup vs baseline: 5.6511x; 1.1846x over previous
import functools

import jax
import jax.numpy as jnp
from jax import lax
from jax.experimental import pallas as pl
from jax.experimental.pallas import tpu as pltpu
from jax.experimental.pallas import tpu_sc as plsc

IGNORE = 255
KEEP_RATIO = 0.3
MIN_KEPT = 100000


_RB = 256


def _ce_body(lg_ref, lb_ref, out_ref):
    x = lg_ref[0]
    m = jnp.max(x, axis=0)
    e = jnp.exp(x - m[None])
    s = jnp.sum(e, axis=0)
    lse = jnp.log(s) + m
    lbl = lb_ref[0]
    ids = lax.broadcasted_iota(jnp.int32, x.shape, 0)
    xl = jnp.sum(jnp.where(ids == lbl[None], x, 0.0), axis=0)
    out_ref[0] = lse - xl


def _ce_loss(logits, labels):
    B, C, H, W = logits.shape
    rows = H * W // 128
    lg = logits.reshape(B, C, rows, 128)
    lb = labels.reshape(B, rows, 128)
    grid = (B, rows // _RB)
    out = pl.pallas_call(
        _ce_body,
        grid=grid,
        in_specs=[
            pl.BlockSpec((1, C, _RB, 128), lambda b, r: (b, 0, r, 0)),
            pl.BlockSpec((1, _RB, 128), lambda b, r: (b, r, 0)),
        ],
        out_specs=pl.BlockSpec((1, _RB, 128), lambda b, r: (b, r, 0)),
        out_shape=jax.ShapeDtypeStruct((B, rows, 128), jnp.float32),
        compiler_params=pltpu.CompilerParams(
            dimension_semantics=("parallel", "parallel")),
    )(lg, lb)
    return out.reshape(B * H * W)


_T = 16
_CH = 8192


def _sc_select(loss, n, k):
    per_tile = n // _T
    nch = per_tile // _CH
    nv = _CH // 16
    kf = float(k)

    mesh = plsc.VectorSubcoreMesh(
        core_axis_name="c", subcore_axis_name="s", num_cores=1)

    @functools.partial(
        pl.kernel,
        out_type=jax.ShapeDtypeStruct((16,), jnp.float32),
        mesh=mesh,
        compiler_params=pltpu.CompilerParams(needs_layout_passes=False),
        scratch_types=[
            pltpu.VMEM((_CH,), jnp.float32),
            pltpu.VMEM((4096,), jnp.float32),
            pltpu.VMEM((4096,), jnp.float32),
            pltpu.VMEM((4096,), jnp.float32),
            pltpu.VMEM((256,), jnp.float32),
            pltpu.VMEM((544,), jnp.float32),
            pltpu.VMEM((256,), jnp.float32),
            pltpu.VMEM((256,), jnp.float32),
            pltpu.VMEM((16,), jnp.float32),
            pltpu.VMEM((16,), jnp.float32),
            pltpu.VMEM((16,), jnp.float32),
            pltpu.VMEM_SHARED((65536,), jnp.float32),
            pltpu.VMEM_SHARED((4096,), jnp.float32),
        ],
    )
    def sel(loss_hbm, out_hbm, buf, hist, totb, mbuf, stripe, mrow,
            cnt3, sum3, stage, asum, acnt, sh_all, sh_tot):
        sid = lax.axis_index("s")
        lane = lax.broadcasted_iota(jnp.int32, (16,), 0)
        ones = jnp.ones((16,), jnp.float32)
        zeros = jnp.zeros((16,), jnp.float32)
        f0 = jnp.float32(0.0)

        def zero_ref(ref, nvec):
            def zb(i, _):
                ref[pl.ds(i * 16, 16)] = zeros
                return 0
            lax.fori_loop(0, nvec, zb, 0)

        def data_pass(body_vec):
            def chunk(c, _):
                base = sid * per_tile + c * _CH
                pltpu.sync_copy(loss_hbm.at[pl.ds(base, _CH)], buf)

                def vec(j, _):
                    v = buf[pl.ds(j * 16, 16)]
                    u = lax.bitcast_convert_type(v, jnp.uint32)
                    body_vec(u, v)
                    return 0
                lax.fori_loop(0, nv, vec, 0)
                return 0
            lax.fori_loop(0, nch, chunk, 0)

        def merge_hist():
            pltpu.sync_copy(hist, sh_all.at[pl.ds(sid * 4096, 4096)])
            plsc.subcore_barrier()
            for src in range(16):
                pltpu.sync_copy(
                    sh_all.at[pl.ds(src * 4096 + sid * 256, 256)],
                    mbuf.at[pl.ds(src * 256, 256)])
            for vb in range(16):
                acc = zeros
                for src in range(16):
                    acc = acc + mbuf[pl.ds(src * 256 + vb * 16, 16)]
                stripe[pl.ds(vb * 16, 16)] = acc
            pltpu.sync_copy(stripe, sh_tot.at[pl.ds(sid * 256, 256)])
            plsc.subcore_barrier()
            pltpu.sync_copy(sh_tot, totb)

        def scan_desc(ref, ngroups, k_rem):
            def body(i, st):
                cum, bstar, above = st
                g = ngroups - 1 - i
                c = ref[pl.ds(g * 16, 16)]
                r = lax.rev(c, (0,))
                cr = jnp.cumsum(r)
                mrk = (cum + cr) >= k_rem
                pc = jnp.max(plsc.all_reduce_population_count(mrk))
                jstar = 16 - pc
                sel_v = jnp.where(lane == jstar, 1.0, 0.0)
                crj = jnp.sum(cr * sel_v)
                rj = jnp.sum(r * sel_v)
                cum_new = cum + jnp.sum(c)
                cond = (bstar < 0.0) & (cum_new >= k_rem)
                bval = (g * 16 + 15 - jstar).astype(jnp.float32)
                bstar = jnp.where(cond, bval, bstar)
                above = jnp.where(cond, cum + crj - rj, above)
                return (cum_new, bstar, above)
            _, bstar, above = lax.fori_loop(
                0, ngroups, body, (f0, jnp.float32(-1.0), f0))
            return bstar, above

        zero_ref(hist, 256)

        def l1(u, v):
            b = (u >> 20).astype(jnp.int32)
            plsc.addupdate_scatter(hist, [b], ones)
        data_pass(l1)
        merge_hist()
        b1, above1 = scan_desc(totb, 256, jnp.float32(kf))
        k2 = kf - above1
        b1u = b1.astype(jnp.int32).astype(jnp.uint32)

        zero_ref(hist, 256)

        def l2(u, v):
            pm = (u >> 20) == b1u
            b = ((u >> 8) & 0xFFF).astype(jnp.int32)
            plsc.addupdate_scatter(hist, [b], ones, mask=pm)
        data_pass(l2)
        merge_hist()
        b2, above2 = scan_desc(totb, 256, k2)
        k3 = k2 - above2
        p24u = (b1.astype(jnp.int32) * 4096
                + b2.astype(jnp.int32)).astype(jnp.uint32)

        zero_ref(cnt3, 16)
        zero_ref(sum3, 16)
        asum[...] = zeros
        acnt[...] = zeros

        def l3(u, v):
            hi = u >> 8
            pm = hi == p24u
            strict = hi > p24u
            b = (u & 0xFF).astype(jnp.int32)
            plsc.addupdate_scatter(cnt3, [b], ones, mask=pm)
            plsc.addupdate_scatter(sum3, [b], v, mask=pm)
            asum[...] = asum[...] + jnp.where(strict, v, 0.0)
            acnt[...] = acnt[...] + jnp.where(strict, ones, zeros)
        data_pass(l3)
        pltpu.sync_copy(cnt3, sh_all.at[pl.ds(sid * 4096, 256)])
        pltpu.sync_copy(sum3, sh_all.at[pl.ds(sid * 4096 + 256, 256)])
        pltpu.sync_copy(asum, sh_all.at[pl.ds(sid * 4096 + 512, 16)])
        pltpu.sync_copy(acnt, sh_all.at[pl.ds(sid * 4096 + 528, 16)])
        plsc.subcore_barrier()

        @pl.when(sid == 0)
        def _():
            zero_ref(cnt3, 16)
            zero_ref(sum3, 16)
            av = zeros
            cv = zeros
            for src in range(16):
                pltpu.sync_copy(sh_all.at[pl.ds(src * 4096, 544)], mrow)
                for vb in range(16):
                    cnt3[pl.ds(vb * 16, 16)] = (
                        cnt3[pl.ds(vb * 16, 16)] + mrow[pl.ds(vb * 16, 16)])
                    sum3[pl.ds(vb * 16, 16)] = (
                        sum3[pl.ds(vb * 16, 16)]
                        + mrow[pl.ds(256 + vb * 16, 16)])
                av = av + mrow[pl.ds(512, 16)]
                cv = cv + mrow[pl.ds(528, 16)]
            b3, _unused = scan_desc(cnt3, 16, k3)
            b3i = b3.astype(jnp.int32)
            sc_v = zeros
            ss_v = zeros
            for i in range(16):
                ge = (i * 16 + lane) >= b3i
                sc_v = sc_v + jnp.where(ge, cnt3[pl.ds(i * 16, 16)], 0.0)
                ss_v = ss_v + jnp.where(ge, sum3[pl.ds(i * 16, 16)], 0.0)
            tot_c = jnp.sum(sc_v) + jnp.sum(cv)
            tot_s = jnp.sum(ss_v) + jnp.sum(av)
            stage[...] = (ones * tot_s) / (ones * tot_c)
            pltpu.sync_copy(stage, out_hbm)

    return sel(loss)


def kernel(logits, labels):
    B, C, H, W = logits.shape
    n = B * H * W
    k = min(max(int(KEEP_RATIO * n), min(MIN_KEPT, n)), n)
    loss = _ce_loss(logits, labels)
    out16 = _sc_select(loss, n, k)
    return out16[0]


# speedup vs baseline: 6.4409x; 1.1398x over previous
import functools

import jax
import jax.numpy as jnp
from jax import lax
from jax.experimental import pallas as pl
from jax.experimental.pallas import tpu as pltpu
from jax.experimental.pallas import tpu_sc as plsc

IGNORE = 255
KEEP_RATIO = 0.3
MIN_KEPT = 100000


_RB = 256


def _ce_body(lg_ref, lb_ref, out_ref):
    x = lg_ref[0]
    m = jnp.max(x, axis=0)
    e = jnp.exp(x - m[None])
    s = jnp.sum(e, axis=0)
    lse = jnp.log(s) + m
    lbl = lb_ref[0]
    ids = lax.broadcasted_iota(jnp.int32, x.shape, 0)
    xl = jnp.sum(jnp.where(ids == lbl[None], x, 0.0), axis=0)
    out_ref[0] = lse - xl


def _ce_loss(logits, labels):
    B, C, H, W = logits.shape
    rows = H * W // 128
    lg = logits.reshape(B, C, rows, 128)
    lb = labels.reshape(B, rows, 128)
    grid = (B, rows // _RB)
    out = pl.pallas_call(
        _ce_body,
        grid=grid,
        in_specs=[
            pl.BlockSpec((1, C, _RB, 128), lambda b, r: (b, 0, r, 0)),
            pl.BlockSpec((1, _RB, 128), lambda b, r: (b, r, 0)),
        ],
        out_specs=pl.BlockSpec((1, _RB, 128), lambda b, r: (b, r, 0)),
        out_shape=jax.ShapeDtypeStruct((B, rows, 128), jnp.float32),
        compiler_params=pltpu.CompilerParams(
            dimension_semantics=("parallel", "parallel")),
    )(lg, lb)
    return out.reshape(B * H * W)


_T = 16
_CH = 32768
_UN = 8


def _sc_select(loss, n, k):
    per_tile = n // _T
    nch = per_tile // _CH
    nv = _CH // 16
    kf = float(k)

    mesh = plsc.VectorSubcoreMesh(
        core_axis_name="c", subcore_axis_name="s", num_cores=1)

    @functools.partial(
        pl.kernel,
        out_type=jax.ShapeDtypeStruct((16,), jnp.float32),
        mesh=mesh,
        compiler_params=pltpu.CompilerParams(needs_layout_passes=False),
        scratch_types=[
            pltpu.VMEM((_CH,), jnp.float32),
            pltpu.VMEM((_CH,), jnp.float32),
            pltpu.SemaphoreType.DMA,
            pltpu.SemaphoreType.DMA,
            pltpu.VMEM((4096,), jnp.float32),
            pltpu.VMEM((4096,), jnp.float32),
            pltpu.VMEM((4096,), jnp.float32),
            pltpu.VMEM((256,), jnp.float32),
            pltpu.VMEM((544,), jnp.float32),
            pltpu.VMEM((256,), jnp.float32),
            pltpu.VMEM((256,), jnp.float32),
            pltpu.VMEM((16,), jnp.float32),
            pltpu.VMEM((16,), jnp.float32),
            pltpu.VMEM((16,), jnp.float32),
            pltpu.VMEM_SHARED((65536,), jnp.float32),
            pltpu.VMEM_SHARED((4096,), jnp.float32),
        ],
    )
    def sel(loss_hbm, out_hbm, buf_a, buf_b, sem_a, sem_b, hist, totb,
            mbuf, stripe, mrow, cnt3, sum3, stage, asum, acnt,
            sh_all, sh_tot):
        sid = lax.axis_index("s")
        lane = lax.broadcasted_iota(jnp.int32, (16,), 0)
        ones = jnp.ones((16,), jnp.float32)
        zeros = jnp.zeros((16,), jnp.float32)
        f0 = jnp.float32(0.0)

        def zero_ref(ref, nvec):
            def zb(i, _):
                ref[pl.ds(i * 16, 16)] = zeros
                return 0
            lax.fori_loop(0, nvec, zb, 0)

        bufs = (buf_a, buf_b)
        sems = (sem_a, sem_b)
        base0 = sid * per_tile

        def data_pass(body_group):
            cps = [None, None]
            cps[0] = pltpu.async_copy(
                loss_hbm.at[pl.ds(base0, _CH)], bufs[0], sems[0])
            for c in range(nch):
                cur = c % 2
                cps[cur].wait()
                if c + 1 < nch:
                    nxt = (c + 1) % 2
                    cps[nxt] = pltpu.async_copy(
                        loss_hbm.at[pl.ds(base0 + (c + 1) * _CH, _CH)],
                        bufs[nxt], sems[nxt])
                b_ref = bufs[cur]

                def it(j, _, b_ref=b_ref):
                    body_group(b_ref, j * (16 * _UN))
                    return 0
                lax.fori_loop(0, nv // _UN, it, 0)

        def merge_hist():
            pltpu.sync_copy(hist, sh_all.at[pl.ds(sid * 4096, 4096)])
            plsc.subcore_barrier()
            for src in range(16):
                pltpu.sync_copy(
                    sh_all.at[pl.ds(src * 4096 + sid * 256, 256)],
                    mbuf.at[pl.ds(src * 256, 256)])
            for vb in range(16):
                acc = zeros
                for src in range(16):
                    acc = acc + mbuf[pl.ds(src * 256 + vb * 16, 16)]
                stripe[pl.ds(vb * 16, 16)] = acc
            pltpu.sync_copy(stripe, sh_tot.at[pl.ds(sid * 256, 256)])
            plsc.subcore_barrier()
            pltpu.sync_copy(sh_tot, totb)

        def scan_desc(ref, ngroups, k_rem):
            def body(i, st):
                cum, bstar, above = st
                g = ngroups - 1 - i
                c = ref[pl.ds(g * 16, 16)]
                r = lax.rev(c, (0,))
                cr = jnp.cumsum(r)
                mrk = (cum + cr) >= k_rem
                pc = jnp.max(plsc.all_reduce_population_count(mrk))
                jstar = 16 - pc
                sel_v = jnp.where(lane == jstar, 1.0, 0.0)
                crj = jnp.sum(cr * sel_v)
                rj = jnp.sum(r * sel_v)
                cum_new = cum + jnp.sum(c)
                cond = (bstar < 0.0) & (cum_new >= k_rem)
                bval = (g * 16 + 15 - jstar).astype(jnp.float32)
                bstar = jnp.where(cond, bval, bstar)
                above = jnp.where(cond, cum + crj - rj, above)
                return (cum_new, bstar, above)
            _, bstar, above = lax.fori_loop(
                0, ngroups, body, (f0, jnp.float32(-1.0), f0))
            return bstar, above

        zero_ref(hist, 256)

        def l1(b_ref, off):
            for t in range(_UN):
                v = b_ref[pl.ds(off + t * 16, 16)]
                u = lax.bitcast_convert_type(v, jnp.uint32)
                b = (u >> 20).astype(jnp.int32)
                plsc.addupdate_scatter(hist, [b], ones)
        data_pass(l1)
        merge_hist()
        b1, above1 = scan_desc(totb, 256, jnp.float32(kf))
        k2 = kf - above1
        b1u = b1.astype(jnp.int32).astype(jnp.uint32)

        zero_ref(hist, 256)

        def l2(b_ref, off):
            for t in range(_UN):
                v = b_ref[pl.ds(off + t * 16, 16)]
                u = lax.bitcast_convert_type(v, jnp.uint32)
                pm = (u >> 20) == b1u
                b = ((u >> 8) & 0xFFF).astype(jnp.int32)
                plsc.addupdate_scatter(hist, [b], ones, mask=pm)
        data_pass(l2)
        merge_hist()
        b2, above2 = scan_desc(totb, 256, k2)
        k3 = k2 - above2
        p24u = (b1.astype(jnp.int32) * 4096
                + b2.astype(jnp.int32)).astype(jnp.uint32)

        zero_ref(cnt3, 16)
        zero_ref(sum3, 16)
        asum[...] = zeros
        acnt[...] = zeros

        def l3(b_ref, off):
            av = zeros
            cv = zeros
            for t in range(_UN):
                v = b_ref[pl.ds(off + t * 16, 16)]
                u = lax.bitcast_convert_type(v, jnp.uint32)
                hi = u >> 8
                pm = hi == p24u
                strict = hi > p24u
                b = (u & 0xFF).astype(jnp.int32)
                plsc.addupdate_scatter(cnt3, [b], ones, mask=pm)
                plsc.addupdate_scatter(sum3, [b], v, mask=pm)
                av = av + jnp.where(strict, v, 0.0)
                cv = cv + jnp.where(strict, ones, zeros)
            asum[...] = asum[...] + av
            acnt[...] = acnt[...] + cv
        data_pass(l3)
        pltpu.sync_copy(cnt3, sh_all.at[pl.ds(sid * 4096, 256)])
        pltpu.sync_copy(sum3, sh_all.at[pl.ds(sid * 4096 + 256, 256)])
        pltpu.sync_copy(asum, sh_all.at[pl.ds(sid * 4096 + 512, 16)])
        pltpu.sync_copy(acnt, sh_all.at[pl.ds(sid * 4096 + 528, 16)])
        plsc.subcore_barrier()

        @pl.when(sid == 0)
        def _():
            zero_ref(cnt3, 16)
            zero_ref(sum3, 16)
            av = zeros
            cv = zeros
            for src in range(16):
                pltpu.sync_copy(sh_all.at[pl.ds(src * 4096, 544)], mrow)
                for vb in range(16):
                    cnt3[pl.ds(vb * 16, 16)] = (
                        cnt3[pl.ds(vb * 16, 16)] + mrow[pl.ds(vb * 16, 16)])
                    sum3[pl.ds(vb * 16, 16)] = (
                        sum3[pl.ds(vb * 16, 16)]
                        + mrow[pl.ds(256 + vb * 16, 16)])
                av = av + mrow[pl.ds(512, 16)]
                cv = cv + mrow[pl.ds(528, 16)]
            b3, _unused = scan_desc(cnt3, 16, k3)
            b3i = b3.astype(jnp.int32)
            sc_v = zeros
            ss_v = zeros
            for i in range(16):
                ge = (i * 16 + lane) >= b3i
                sc_v = sc_v + jnp.where(ge, cnt3[pl.ds(i * 16, 16)], 0.0)
                ss_v = ss_v + jnp.where(ge, sum3[pl.ds(i * 16, 16)], 0.0)
            tot_c = jnp.sum(sc_v) + jnp.sum(cv)
            tot_s = jnp.sum(ss_v) + jnp.sum(av)
            stage[...] = (ones * tot_s) / (ones * tot_c)
            pltpu.sync_copy(stage, out_hbm)

    return sel(loss)


def kernel(logits, labels):
    B, C, H, W = logits.shape
    n = B * H * W
    k = min(max(int(KEEP_RATIO * n), min(MIN_KEPT, n)), n)
    loss = _ce_loss(logits, labels)
    out16 = _sc_select(loss, n, k)
    return out16[0]


# speedup vs baseline: 9.2259x; 1.4324x over previous
import functools

import jax
import jax.numpy as jnp
from jax import lax
from jax.experimental import pallas as pl
from jax.experimental.pallas import tpu as pltpu
from jax.experimental.pallas import tpu_sc as plsc

IGNORE = 255
KEEP_RATIO = 0.3
MIN_KEPT = 100000


_RB = 256


def _ce_body(lg_ref, lb_ref, out_ref):
    x = lg_ref[0]
    m = jnp.max(x, axis=0)
    e = jnp.exp(x - m[None])
    s = jnp.sum(e, axis=0)
    lse = jnp.log(s) + m
    lbl = lb_ref[0]
    ids = lax.broadcasted_iota(jnp.int32, x.shape, 0)
    xl = jnp.sum(jnp.where(ids == lbl[None], x, 0.0), axis=0)
    out_ref[0] = lse - xl


def _ce_loss(logits, labels):
    B, C, H, W = logits.shape
    rows = H * W // 128
    lg = logits.reshape(B, C, rows, 128)
    lb = labels.reshape(B, rows, 128)
    grid = (B, rows // _RB)
    out = pl.pallas_call(
        _ce_body,
        grid=grid,
        in_specs=[
            pl.BlockSpec((1, C, _RB, 128), lambda b, r: (b, 0, r, 0)),
            pl.BlockSpec((1, _RB, 128), lambda b, r: (b, r, 0)),
        ],
        out_specs=pl.BlockSpec((1, _RB, 128), lambda b, r: (b, r, 0)),
        out_shape=jax.ShapeDtypeStruct((B, rows, 128), jnp.float32),
        compiler_params=pltpu.CompilerParams(
            dimension_semantics=("parallel", "parallel")),
    )(lg, lb)
    return out.reshape(B * H * W)


_T = 16
_CH = 32768
_UN = 8


def _sc_select(loss, n, k):
    per_tile = n // _T
    nch = per_tile // _CH
    nv = _CH // 16
    kf = float(k)

    mesh = plsc.VectorSubcoreMesh(
        core_axis_name="c", subcore_axis_name="s", num_cores=1)

    @functools.partial(
        pl.kernel,
        out_type=jax.ShapeDtypeStruct((16,), jnp.float32),
        mesh=mesh,
        compiler_params=pltpu.CompilerParams(needs_layout_passes=False),
        scratch_types=[
            pltpu.VMEM((_CH,), jnp.float32),
            pltpu.VMEM((_CH,), jnp.float32),
            pltpu.SemaphoreType.DMA,
            pltpu.SemaphoreType.DMA,
            pltpu.VMEM((4096,), jnp.float32),
            pltpu.VMEM((4096,), jnp.float32),
            pltpu.VMEM((4096,), jnp.float32),
            pltpu.VMEM((256,), jnp.float32),
            pltpu.VMEM((544,), jnp.float32),
            pltpu.VMEM((256,), jnp.float32),
            pltpu.VMEM((256,), jnp.float32),
            pltpu.VMEM((16,), jnp.float32),
            pltpu.VMEM((16,), jnp.float32),
            pltpu.VMEM((16,), jnp.float32),
            pltpu.VMEM_SHARED((65536,), jnp.float32),
            pltpu.VMEM_SHARED((4096,), jnp.float32),
        ],
    )
    def sel(loss_hbm, out_hbm, buf_a, buf_b, sem_a, sem_b, hist, totb,
            mbuf, stripe, mrow, cnt3, sum3, stage, asum, acnt,
            sh_all, sh_tot):
        sid = lax.axis_index("s")
        lane = lax.broadcasted_iota(jnp.int32, (16,), 0)
        ones = jnp.ones((16,), jnp.float32)
        zeros = jnp.zeros((16,), jnp.float32)
        f0 = jnp.float32(0.0)

        def zero_ref(ref, nvec):
            def zb(i, _):
                ref[pl.ds(i * 16, 16)] = zeros
                return 0
            lax.fori_loop(0, nvec, zb, 0)

        bufs = (buf_a, buf_b)
        sems = (sem_a, sem_b)
        base0 = sid * per_tile

        def data_pass(body_group):
            cps = [None, None]
            cps[0] = pltpu.async_copy(
                loss_hbm.at[pl.ds(base0, _CH)], bufs[0], sems[0])
            for c in range(nch):
                cur = c % 2
                cps[cur].wait()
                if c + 1 < nch:
                    nxt = (c + 1) % 2
                    cps[nxt] = pltpu.async_copy(
                        loss_hbm.at[pl.ds(base0 + (c + 1) * _CH, _CH)],
                        bufs[nxt], sems[nxt])
                body_group(bufs[cur])

        def merge_hist():
            pltpu.sync_copy(hist, sh_all.at[pl.ds(sid * 4096, 4096)])
            plsc.subcore_barrier()
            for src in range(16):
                pltpu.sync_copy(
                    sh_all.at[pl.ds(src * 4096 + sid * 256, 256)],
                    mbuf.at[pl.ds(src * 256, 256)])
            for vb in range(16):
                acc = zeros
                for src in range(16):
                    acc = acc + mbuf[pl.ds(src * 256 + vb * 16, 16)]
                stripe[pl.ds(vb * 16, 16)] = acc
            pltpu.sync_copy(stripe, sh_tot.at[pl.ds(sid * 256, 256)])
            plsc.subcore_barrier()
            pltpu.sync_copy(sh_tot, totb)

        def scan_desc(ref, ngroups, k_rem):
            def body(i, st):
                cum, bstar, above = st
                g = ngroups - 1 - i
                c = ref[pl.ds(g * 16, 16)]
                r = lax.rev(c, (0,))
                cr = jnp.cumsum(r)
                mrk = (cum + cr) >= k_rem
                pc = jnp.max(plsc.all_reduce_population_count(mrk))
                jstar = 16 - pc
                sel_v = jnp.where(lane == jstar, 1.0, 0.0)
                crj = jnp.sum(cr * sel_v)
                rj = jnp.sum(r * sel_v)
                cum_new = cum + jnp.sum(c)
                cond = (bstar < 0.0) & (cum_new >= k_rem)
                bval = (g * 16 + 15 - jstar).astype(jnp.float32)
                bstar = jnp.where(cond, bval, bstar)
                above = jnp.where(cond, cum + crj - rj, above)
                return (cum_new, bstar, above)
            _, bstar, above = lax.fori_loop(
                0, ngroups, body, (f0, jnp.float32(-1.0), f0))
            return bstar, above

        zero_ref(hist, 256)

        def l1(b_ref):
            @plsc.parallel_loop(0, _CH, 16, unroll=_UN)
            def _(i):
                v = b_ref[pl.ds(i, 16)]
                u = lax.bitcast_convert_type(v, jnp.uint32)
                b = (u >> 20).astype(jnp.int32)
                plsc.addupdate_scatter(hist, [b], ones)
        data_pass(l1)
        merge_hist()
        b1, above1 = scan_desc(totb, 256, jnp.float32(kf))
        k2 = kf - above1
        b1u = b1.astype(jnp.int32).astype(jnp.uint32)

        zero_ref(hist, 256)

        def l2(b_ref):
            @plsc.parallel_loop(0, _CH, 16, unroll=_UN)
            def _(i):
                v = b_ref[pl.ds(i, 16)]
                u = lax.bitcast_convert_type(v, jnp.uint32)
                pm = (u >> 20) == b1u
                b = ((u >> 8) & 0xFFF).astype(jnp.int32)
                plsc.addupdate_scatter(hist, [b], ones, mask=pm)
        data_pass(l2)
        merge_hist()
        b2, above2 = scan_desc(totb, 256, k2)
        k3 = k2 - above2
        p24u = (b1.astype(jnp.int32) * 4096
                + b2.astype(jnp.int32)).astype(jnp.uint32)

        zero_ref(cnt3, 16)
        zero_ref(sum3, 16)
        asum[...] = zeros
        acnt[...] = zeros

        def l3(b_ref):
            @plsc.parallel_loop(0, _CH, 16, unroll=_UN, carry=(zeros, zeros))
            def acc(i, cr):
                av, cv = cr
                v = b_ref[pl.ds(i, 16)]
                u = lax.bitcast_convert_type(v, jnp.uint32)
                hi = u >> 8
                pm = hi == p24u
                strict = hi > p24u
                b = (u & 0xFF).astype(jnp.int32)
                plsc.addupdate_scatter(cnt3, [b], ones, mask=pm)
                plsc.addupdate_scatter(sum3, [b], v, mask=pm)
                av = av + jnp.where(strict, v, 0.0)
                cv = cv + jnp.where(strict, ones, zeros)
                return (av, cv)
            av, cv = acc
            asum[...] = asum[...] + av
            acnt[...] = acnt[...] + cv
        data_pass(l3)
        pltpu.sync_copy(cnt3, sh_all.at[pl.ds(sid * 4096, 256)])
        pltpu.sync_copy(sum3, sh_all.at[pl.ds(sid * 4096 + 256, 256)])
        pltpu.sync_copy(asum, sh_all.at[pl.ds(sid * 4096 + 512, 16)])
        pltpu.sync_copy(acnt, sh_all.at[pl.ds(sid * 4096 + 528, 16)])
        plsc.subcore_barrier()

        @pl.when(sid == 0)
        def _():
            zero_ref(cnt3, 16)
            zero_ref(sum3, 16)
            av = zeros
            cv = zeros
            for src in range(16):
                pltpu.sync_copy(sh_all.at[pl.ds(src * 4096, 544)], mrow)
                for vb in range(16):
                    cnt3[pl.ds(vb * 16, 16)] = (
                        cnt3[pl.ds(vb * 16, 16)] + mrow[pl.ds(vb * 16, 16)])
                    sum3[pl.ds(vb * 16, 16)] = (
                        sum3[pl.ds(vb * 16, 16)]
                        + mrow[pl.ds(256 + vb * 16, 16)])
                av = av + mrow[pl.ds(512, 16)]
                cv = cv + mrow[pl.ds(528, 16)]
            b3, _unused = scan_desc(cnt3, 16, k3)
            b3i = b3.astype(jnp.int32)
            sc_v = zeros
            ss_v = zeros
            for i in range(16):
                ge = (i * 16 + lane) >= b3i
                sc_v = sc_v + jnp.where(ge, cnt3[pl.ds(i * 16, 16)], 0.0)
                ss_v = ss_v + jnp.where(ge, sum3[pl.ds(i * 16, 16)], 0.0)
            tot_c = jnp.sum(sc_v) + jnp.sum(cv)
            tot_s = jnp.sum(ss_v) + jnp.sum(av)
            stage[...] = (ones * tot_s) / (ones * tot_c)
            pltpu.sync_copy(stage, out_hbm)

    return sel(loss)


def kernel(logits, labels):
    B, C, H, W = logits.shape
    n = B * H * W
    k = min(max(int(KEEP_RATIO * n), min(MIN_KEPT, n)), n)
    loss = _ce_loss(logits, labels)
    out16 = _sc_select(loss, n, k)
    return out16[0]


# speedup vs baseline: 9.2619x; 1.0039x over previous
import functools

import jax
import jax.numpy as jnp
from jax import lax
from jax.experimental import pallas as pl
from jax.experimental.pallas import tpu as pltpu
from jax.experimental.pallas import tpu_sc as plsc

IGNORE = 255
KEEP_RATIO = 0.3
MIN_KEPT = 100000

_NQ = 4
_RB = 256
_T = 16
_CH = 32768
_UN = 8


def _ce_body(lg_ref, lb_ref, out_ref):
    x = lg_ref[0]
    m = jnp.max(x, axis=0)
    e = jnp.exp(x - m[None])
    s = jnp.sum(e, axis=0)
    lse = jnp.log(s) + m
    lbl = lb_ref[0]
    ids = lax.broadcasted_iota(jnp.int32, x.shape, 0)
    xl = jnp.sum(jnp.where(ids == lbl[None], x, 0.0), axis=0)
    out_ref[0] = lse - xl


def _ce_loss_quarter(logits, labels, q, bq):
    B, C, H, W = logits.shape
    rows = H * W // 128
    lg = logits.reshape(B, C, rows, 128)
    lb = labels.reshape(B, rows, 128)
    grid = (bq, rows // _RB)
    out = pl.pallas_call(
        _ce_body,
        grid=grid,
        in_specs=[
            pl.BlockSpec((1, C, _RB, 128), lambda b, r: (q * bq + b, 0, r, 0)),
            pl.BlockSpec((1, _RB, 128), lambda b, r: (q * bq + b, r, 0)),
        ],
        out_specs=pl.BlockSpec((1, _RB, 128), lambda b, r: (b, r, 0)),
        out_shape=jax.ShapeDtypeStruct((bq, rows, 128), jnp.float32),
        compiler_params=pltpu.CompilerParams(
            dimension_semantics=("parallel", "parallel")),
    )(lg, lb)
    return out.reshape(bq * H * W)


def _sc_hist12(loss_q, nq_elems):
    per_tile = nq_elems // (2 * _T)

    mesh = plsc.VectorSubcoreMesh(
        core_axis_name="c", subcore_axis_name="s", num_cores=2)

    @functools.partial(
        pl.kernel,
        out_type=jax.ShapeDtypeStruct((2 * 4096,), jnp.float32),
        mesh=mesh,
        compiler_params=pltpu.CompilerParams(needs_layout_passes=False),
        scratch_types=[
            pltpu.VMEM((per_tile,), jnp.float32),
            pltpu.VMEM((4096,), jnp.float32),
            pltpu.VMEM((4096,), jnp.float32),
            pltpu.VMEM((256,), jnp.float32),
            pltpu.VMEM_SHARED((65536,), jnp.float32),
        ],
    )
    def h12(loss_hbm, out_hbm, buf, hist, mbuf, stripe, sh_all):
        cid = lax.axis_index("c")
        sid = lax.axis_index("s")
        wid = cid * _T + sid
        ones = jnp.ones((16,), jnp.float32)
        zeros = jnp.zeros((16,), jnp.float32)

        def zb(i, _):
            hist[pl.ds(i * 16, 16)] = zeros
            return 0
        lax.fori_loop(0, 256, zb, 0)

        pltpu.sync_copy(loss_hbm.at[pl.ds(wid * per_tile, per_tile)], buf)

        @plsc.parallel_loop(0, per_tile, 16, unroll=_UN)
        def _(i):
            v = buf[pl.ds(i, 16)]
            u = lax.bitcast_convert_type(v, jnp.uint32)
            b = (u >> 20).astype(jnp.int32)
            plsc.addupdate_scatter(hist, [b], ones)

        pltpu.sync_copy(hist, sh_all.at[pl.ds(sid * 4096, 4096)])
        plsc.subcore_barrier()
        for src in range(16):
            pltpu.sync_copy(sh_all.at[pl.ds(src * 4096 + sid * 256, 256)],
                            mbuf.at[pl.ds(src * 256, 256)])
        for vb in range(16):
            acc = zeros
            for src in range(16):
                acc = acc + mbuf[pl.ds(src * 256 + vb * 16, 16)]
            stripe[pl.ds(vb * 16, 16)] = acc
        pltpu.sync_copy(stripe,
                        out_hbm.at[pl.ds(cid * 4096 + sid * 256, 256)])

    return h12(loss_q)


def _sc_select(losses, hists, n, k):
    per_q = n // _NQ
    share = per_q // _T
    kf = float(k)

    mesh = plsc.VectorSubcoreMesh(
        core_axis_name="c", subcore_axis_name="s", num_cores=1)

    @functools.partial(
        pl.kernel,
        out_type=jax.ShapeDtypeStruct((16,), jnp.float32),
        mesh=mesh,
        compiler_params=pltpu.CompilerParams(needs_layout_passes=False),
        scratch_types=[
            pltpu.VMEM((_CH,), jnp.float32),
            pltpu.VMEM((_CH,), jnp.float32),
            pltpu.SemaphoreType.DMA,
            pltpu.SemaphoreType.DMA,
            pltpu.VMEM((8192,), jnp.float32),
            pltpu.VMEM((4096,), jnp.float32),
            pltpu.VMEM((4096,), jnp.float32),
            pltpu.VMEM((4096,), jnp.float32),
            pltpu.VMEM((256,), jnp.float32),
            pltpu.VMEM((544,), jnp.float32),
            pltpu.VMEM((256,), jnp.float32),
            pltpu.VMEM((16,), jnp.float32),
            pltpu.VMEM((16,), jnp.float32),
            pltpu.VMEM((16,), jnp.float32),
            pltpu.VMEM_SHARED((65536,), jnp.float32),
            pltpu.VMEM_SHARED((4096,), jnp.float32),
        ],
    )
    def sel(l0, l1_, l2_, l3_, h0, h1_, h2_, h3_, out_hbm,
            buf_a, buf_b, sem_a, sem_b, qh, hist, totb, mbuf, stripe, mrow,
            cnt3, stage, asum, acnt, sh_all, sh_tot):
        sid = lax.axis_index("s")
        lane = lax.broadcasted_iota(jnp.int32, (16,), 0)
        ones = jnp.ones((16,), jnp.float32)
        zeros = jnp.zeros((16,), jnp.float32)
        f0 = jnp.float32(0.0)
        qlosses = (l0, l1_, l2_, l3_)
        qhists = (h0, h1_, h2_, h3_)

        def zero_ref(ref, nvec):
            def zb(i, _):
                ref[pl.ds(i * 16, 16)] = zeros
                return 0
            lax.fori_loop(0, nvec, zb, 0)

        bufs = (buf_a, buf_b)
        sems = (sem_a, sem_b)

        def data_pass(body_group):
            cps = [None, None]
            cps[0] = pltpu.async_copy(
                qlosses[0].at[pl.ds(sid * share, share)], bufs[0], sems[0])
            for c in range(_NQ):
                cur = c % 2
                cps[cur].wait()
                if c + 1 < _NQ:
                    nxt = (c + 1) % 2
                    cps[nxt] = pltpu.async_copy(
                        qlosses[c + 1].at[pl.ds(sid * share, share)],
                        bufs[nxt], sems[nxt])
                body_group(bufs[cur])

        def merge_hist():
            pltpu.sync_copy(hist, sh_all.at[pl.ds(sid * 4096, 4096)])
            plsc.subcore_barrier()
            for src in range(16):
                pltpu.sync_copy(
                    sh_all.at[pl.ds(src * 4096 + sid * 256, 256)],
                    mbuf.at[pl.ds(src * 256, 256)])
            for vb in range(16):
                acc = zeros
                for src in range(16):
                    acc = acc + mbuf[pl.ds(src * 256 + vb * 16, 16)]
                stripe[pl.ds(vb * 16, 16)] = acc
            pltpu.sync_copy(stripe, sh_tot.at[pl.ds(sid * 256, 256)])
            plsc.subcore_barrier()
            pltpu.sync_copy(sh_tot, totb)

        def scan_desc(ref, ngroups, k_rem):
            def body(i, st):
                cum, bstar, above = st
                g = ngroups - 1 - i
                c = ref[pl.ds(g * 16, 16)]
                r = lax.rev(c, (0,))
                cr = jnp.cumsum(r)
                mrk = (cum + cr) >= k_rem
                pc = jnp.max(plsc.all_reduce_population_count(mrk))
                jstar = 16 - pc
                sel_v = jnp.where(lane == jstar, 1.0, 0.0)
                crj = jnp.sum(cr * sel_v)
                rj = jnp.sum(r * sel_v)
                cum_new = cum + jnp.sum(c)
                cond = (bstar < 0.0) & (cum_new >= k_rem)
                bval = (g * 16 + 15 - jstar).astype(jnp.float32)
                bstar = jnp.where(cond, bval, bstar)
                above = jnp.where(cond, cum + crj - rj, above)
                return (cum_new, bstar, above)
            _, bstar, above = lax.fori_loop(
                0, ngroups, body, (f0, jnp.float32(-1.0), f0))
            return bstar, above

        zero_ref(totb, 256)
        for q in range(_NQ):
            pltpu.sync_copy(qhists[q], qh)

            def addq(i, _):
                totb[pl.ds(i * 16, 16)] = (
                    totb[pl.ds(i * 16, 16)] + qh[pl.ds(i * 16, 16)]
                    + qh[pl.ds(4096 + i * 16, 16)])
                return 0
            lax.fori_loop(0, 256, addq, 0)
        b1, above1 = scan_desc(totb, 256, jnp.float32(kf))
        k2 = kf - above1
        b1u = b1.astype(jnp.int32).astype(jnp.uint32)

        zero_ref(hist, 256)

        def l2(b_ref):
            @plsc.parallel_loop(0, _CH, 16, unroll=_UN)
            def _(i):
                v = b_ref[pl.ds(i, 16)]
                u = lax.bitcast_convert_type(v, jnp.uint32)
                pm = (u >> 20) == b1u
                b = ((u >> 8) & 0xFFF).astype(jnp.int32)
                plsc.addupdate_scatter(hist, [b], ones, mask=pm)
        data_pass(l2)
        merge_hist()
        b2, above2 = scan_desc(totb, 256, k2)
        k3 = k2 - above2
        p24u = (b1.astype(jnp.int32) * 4096
                + b2.astype(jnp.int32)).astype(jnp.uint32)

        zero_ref(cnt3, 16)
        asum[...] = zeros
        acnt[...] = zeros

        def l3(b_ref):
            @plsc.parallel_loop(0, _CH, 16, unroll=_UN, carry=(zeros, zeros))
            def acc(i, cr):
                av, cv = cr
                v = b_ref[pl.ds(i, 16)]
                u = lax.bitcast_convert_type(v, jnp.uint32)
                hi = u >> 8
                pm = hi == p24u
                strict = hi > p24u
                b = (u & 0xFF).astype(jnp.int32)
                plsc.addupdate_scatter(cnt3, [b], ones, mask=pm)
                av = av + jnp.where(strict, v, 0.0)
                cv = cv + jnp.where(strict, ones, zeros)
                return (av, cv)
            av, cv = acc
            asum[...] = asum[...] + av
            acnt[...] = acnt[...] + cv
        data_pass(l3)
        pltpu.sync_copy(cnt3, sh_all.at[pl.ds(sid * 4096, 256)])
        pltpu.sync_copy(asum, sh_all.at[pl.ds(sid * 4096 + 512, 16)])
        pltpu.sync_copy(acnt, sh_all.at[pl.ds(sid * 4096 + 528, 16)])
        plsc.subcore_barrier()

        @pl.when(sid == 0)
        def _():
            zero_ref(cnt3, 16)
            av = zeros
            cv = zeros
            for src in range(16):
                pltpu.sync_copy(sh_all.at[pl.ds(src * 4096, 544)], mrow)
                for vb in range(16):
                    cnt3[pl.ds(vb * 16, 16)] = (
                        cnt3[pl.ds(vb * 16, 16)] + mrow[pl.ds(vb * 16, 16)])
                av = av + mrow[pl.ds(512, 16)]
                cv = cv + mrow[pl.ds(528, 16)]
            b3, _unused = scan_desc(cnt3, 16, k3)
            b3i = b3.astype(jnp.int32)
            sc_v = zeros
            ss_v = zeros
            for i in range(16):
                gb = i * 16 + lane
                ge = gb >= b3i
                bits = (p24u << 8) | gb.astype(jnp.uint32)
                val = lax.bitcast_convert_type(bits, jnp.float32)
                cnt_g = cnt3[pl.ds(i * 16, 16)]
                sc_v = sc_v + jnp.where(ge, cnt_g, 0.0)
                ss_v = ss_v + jnp.where(ge, cnt_g * val, 0.0)
            tot_c = jnp.sum(sc_v) + jnp.sum(cv)
            tot_s = jnp.sum(ss_v) + jnp.sum(av)
            stage[...] = (ones * tot_s) / (ones * tot_c)
            pltpu.sync_copy(stage, out_hbm)

    return sel(*losses, *hists)


def kernel(logits, labels):
    B, C, H, W = logits.shape
    n = B * H * W
    k = min(max(int(KEEP_RATIO * n), min(MIN_KEPT, n)), n)
    bq = B // _NQ
    losses = []
    hists = []
    for q in range(_NQ):
        lq = _ce_loss_quarter(logits, labels, q, bq)
        losses.append(lq)
        hists.append(_sc_hist12(lq, bq * H * W))
    out16 = _sc_select(losses, hists, n, k)
    return out16[0]
